# Initial kernel scaffold; baseline (speedup 1.0000x reference)
import jax
import jax.numpy as jnp
from jax.experimental import pallas as pl


def kernel(x, edge_index, edge_feat, lg_edge_index, rel_embed, Wself, bself, Wk, bk, Wfuse, bfuse, Wff1, bff1, Wff2, bff2, ln1_g, ln1_b, ln2_g, ln2_b):
    raise NotImplementedError("write your pallas kernel here")


# speedup vs baseline: 1.1514x; 1.1514x over previous
import functools

import jax
import jax.numpy as jnp
from jax.experimental import pallas as pl
from jax.experimental.pallas import tpu as pltpu

N = 10000
E = 320000
H = 128
K = 4


def _dense_body(xin_ref, emsg_ref, z1_ref, z2_ref, z3_ref, z4_ref,
                ws_ref, bs_ref, wk_ref, bk_ref, wf_ref, bf_ref,
                w1_ref, b1_ref, w2_ref, b2_ref,
                g1_ref, t1_ref, g2_ref, t2_ref, out_ref):
    xin = xin_ref[...]
    acc = jnp.dot(xin, ws_ref[...], preferred_element_type=jnp.float32)
    acc += bs_ref[...]
    acc += jnp.dot(emsg_ref[...], wf_ref[...], preferred_element_type=jnp.float32)
    acc += bf_ref[...]
    zs = (z1_ref, z2_ref, z3_ref, z4_ref)
    for i in range(K):
        acc += jnp.dot(zs[i][...], wk_ref[i], preferred_element_type=jnp.float32)
        acc += bk_ref[i]

    def _gelu(v):
        return 0.5 * v * (1.0 + jax.lax.erf(v * 0.7071067811865476))

    def _ln(v, g, b):
        m = jnp.mean(v, axis=-1, keepdims=True)
        c = v - m
        var = jnp.mean(c * c, axis=-1, keepdims=True)
        return c * jax.lax.rsqrt(var + 1e-5) * g + b

    out1 = _ln(xin + _gelu(acc), g1_ref[...], t1_ref[...])
    ff = jnp.dot(_gelu(jnp.dot(out1, w1_ref[...], preferred_element_type=jnp.float32)
                       + b1_ref[...]),
                 w2_ref[...], preferred_element_type=jnp.float32) + b2_ref[...]
    out_ref[...] = _ln(out1 + ff, g2_ref[...], t2_ref[...])


def _dense_core(xin, emsg, zs, Ws, bs, Wk, bk, Wf, bf, W1, b1, W2, b2, g1, t1, g2, t2):
    n = xin.shape[0]
    BN = 512
    grid = (pl.cdiv(n, BN),)
    row = lambda i: (i, 0)
    const2 = lambda i: (0, 0)
    const3 = lambda i: (0, 0, 0)
    in_specs = [
        pl.BlockSpec((BN, H), row),
        pl.BlockSpec((BN, H), row),
        pl.BlockSpec((BN, H), row),
        pl.BlockSpec((BN, H), row),
        pl.BlockSpec((BN, H), row),
        pl.BlockSpec((BN, H), row),
        pl.BlockSpec((H, H), const2),
        pl.BlockSpec((1, H), const2),
        pl.BlockSpec((K, H, H), const3),
        pl.BlockSpec((K, 1, H), const3),
        pl.BlockSpec((H, H), const2),
        pl.BlockSpec((1, H), const2),
        pl.BlockSpec((H, 4 * H), const2),
        pl.BlockSpec((1, 4 * H), const2),
        pl.BlockSpec((4 * H, H), const2),
        pl.BlockSpec((1, H), const2),
        pl.BlockSpec((1, H), const2),
        pl.BlockSpec((1, H), const2),
        pl.BlockSpec((1, H), const2),
        pl.BlockSpec((1, H), const2),
    ]
    return pl.pallas_call(
        _dense_body,
        grid=grid,
        in_specs=in_specs,
        out_specs=pl.BlockSpec((BN, H), row),
        out_shape=jax.ShapeDtypeStruct((n, H), jnp.float32),
    )(xin, emsg, zs[0], zs[1], zs[2], zs[3],
      Ws, bs.reshape(1, H), Wk, bk.reshape(K, 1, H), Wf, bf.reshape(1, H),
      W1, b1.reshape(1, 4 * H), W2, b2.reshape(1, H),
      g1.reshape(1, H), t1.reshape(1, H), g2.reshape(1, H), t2.reshape(1, H))


def kernel(x, edge_index, edge_feat, lg_edge_index, rel_embed, Wself, bself,
           Wk, bk, Wfuse, bfuse, Wff1, bff1, Wff2, bff2, ln1_g, ln1_b, ln2_g, ln2_b):
    src = edge_index[0]
    dst = edge_index[1]
    lsrc = lg_edge_index[0]
    ldst = lg_edge_index[1]
    lg_x = rel_embed[edge_feat]
    n_nodes = x.shape[0]
    n_edges = src.shape[0]
    xx = x
    for l in range(2):
        node_msg = (jnp.zeros((n_nodes, H), x.dtype)
                    .at[src].add(lg_x).at[dst].add(lg_x))
        emsg = xx[src] + xx[dst]
        zn = xx
        zhops_n = []
        for i in range(K):
            zn = jnp.zeros((n_nodes, H), x.dtype).at[dst].add(zn[src])
            zhops_n.append(zn)
        ze = lg_x
        zhops_e = []
        for i in range(K):
            ze = jnp.zeros((n_edges, H), x.dtype).at[ldst].add(ze[lsrc])
            zhops_e.append(ze)
        c0, c1 = 2 * l, 2 * l + 1
        new_x = _dense_core(xx, node_msg, zhops_n,
                            Wself[c0], bself[c0], Wk[c0], bk[c0], Wfuse[c0],
                            bfuse[c0], Wff1[c0], bff1[c0], Wff2[c0], bff2[c0],
                            ln1_g[c0], ln1_b[c0], ln2_g[c0], ln2_b[c0])
        new_lg = _dense_core(lg_x, emsg, zhops_e,
                             Wself[c1], bself[c1], Wk[c1], bk[c1], Wfuse[c1],
                             bfuse[c1], Wff1[c1], bff1[c1], Wff2[c1], bff2[c1],
                             ln1_g[c1], ln1_b[c1], ln2_g[c1], ln2_b[c1])
        xx = new_x
        lg_x = new_lg
    return (xx, lg_x)
